# baseline (speedup 1.0000x reference)
import jax
import jax.numpy as jnp
from jax import lax
from jax.experimental import pallas as pl
from jax.experimental.pallas import tpu as pltpu
import numpy as np

NUM_IN = 100000
NUM_OUT = 1024
ROWS = 8
SLAB = NUM_OUT * NUM_IN

_TINY = np.float32(np.finfo(np.float32).tiny)
_ONE_BITS = np.uint32(np.float32(1.0).view(np.uint32))

W = 768
NCH = NUM_IN // W
TAIL = NUM_IN - NCH * W

_ROTS = ((13, 15, 26, 6), (17, 29, 16, 24))


def _hash_bits(ks, x1):
    ks0, ks1, ks2 = ks
    x0 = ks0
    inj0 = (ks1, ks2, ks0, ks1, ks2)
    inj1 = (ks2 + np.uint32(1), ks0 + np.uint32(2), ks1 + np.uint32(3),
            ks2 + np.uint32(4), ks0 + np.uint32(5))
    for i in range(5):
        for r in _ROTS[i % 2]:
            x0 = x0 + x1
            x1 = ((x1 << np.uint32(r)) | (x1 >> np.uint32(32 - r))) ^ x0
        x0 = x0 + inj0[i]
        x1 = x1 + inj1[i]
    return x0 ^ x1


def _bits_to_open01(bits):
    fb = (bits >> np.uint32(9)) | _ONE_BITS
    f = lax.bitcast_convert_type(fb, jnp.float32) - jnp.float32(1.0)
    return jnp.maximum(_TINY, f)


def _sample_kernel(keys_ref, temp_ref, adj_ref, nor_ref, out_ref, bern_ref):
    pid = pl.program_id(0)
    t = temp_ref[0]
    k0 = keys_ref[0]
    k1 = keys_ref[1]
    ks = (k0, k1, k0 ^ k1 ^ np.uint32(0x1BD11BDA))

    row0 = (pid * ROWS).astype(jnp.uint32)
    r_u = lax.broadcasted_iota(jnp.uint32, (ROWS, 1), 0) + row0
    base0 = r_u * np.uint32(NUM_IN) + k1

    def vals_for(logits, x1c):
        out = []
        for s in range(2):
            bits = _hash_bits(ks, x1c if s == 0 else x1c + np.uint32(SLAB))
            u = _bits_to_open01(bits)
            out.append(logits - jnp.log(-jnp.log(u)))
        return out

    def body(j, carry):
        x1c, (av0, aj0, av1, aj1) = carry
        logits = adj_ref[:, pl.ds(j * W, W)] * t
        v0, v1 = vals_for(logits, x1c)
        up0 = v0 > av0
        up1 = v1 > av1
        av0 = jnp.where(up0, v0, av0)
        aj0 = jnp.where(up0, j, aj0)
        av1 = jnp.where(up1, v1, av1)
        aj1 = jnp.where(up1, j, aj1)
        return x1c + np.uint32(W), (av0, aj0, av1, aj1)

    c_w = lax.broadcasted_iota(jnp.uint32, (ROWS, W), 1)
    neg_inf = jnp.full((ROWS, W), -jnp.inf, jnp.float32)
    zeros_i = jnp.zeros((ROWS, W), jnp.int32)
    _, (av0, aj0, av1, aj1) = lax.fori_loop(
        0, NCH, body,
        (base0 + c_w, (neg_inf, zeros_i, neg_inf, zeros_i)))

    c_t = lax.broadcasted_iota(jnp.uint32, (ROWS, TAIL), 1)
    t_logits = adj_ref[:, pl.ds(NCH * W, TAIL)] * t
    tv0, tv1 = vals_for(t_logits, base0 + c_t + np.uint32(NCH * W))

    lane_w = lax.broadcasted_iota(jnp.int32, (ROWS, W), 1)
    lane_t = lax.broadcasted_iota(jnp.int32, (ROWS, TAIL), 1)
    sentinel = jnp.int32(np.iinfo(np.int32).max)

    cols = []
    for av, aj, tv in ((av0, aj0, tv0), (av1, aj1, tv1)):
        m_main = jnp.max(av, axis=1, keepdims=True)
        c_main = aj * W + lane_w
        i_main = jnp.min(jnp.where(av == m_main, c_main, sentinel), axis=1)
        m_tail = jnp.max(tv, axis=1, keepdims=True)
        c_tail = lane_t + jnp.int32(NCH * W)
        i_tail = jnp.min(jnp.where(tv == m_tail, c_tail, sentinel), axis=1)
        tw = m_tail[:, 0] > m_main[:, 0]
        cols.append(jnp.where(tw, i_tail, i_main)[:, None])
    out_ref[...] = jnp.concatenate(cols, axis=1)

    @pl.when(pid == 0)
    def _():
        tb = temp_ref[1]
        kb0 = keys_ref[2]
        kb1 = keys_ref[3]
        kb = (kb0, kb1, kb0 ^ kb1 ^ np.uint32(0x1BD11BDA))
        p = jax.nn.sigmoid(nor_ref[...] * tb)
        a = lax.broadcasted_iota(jnp.uint32, (8, 128), 0)
        b = lax.broadcasted_iota(jnp.uint32, (8, 128), 1)
        x1 = a * np.uint32(128) + b + kb1
        bits = _hash_bits(kb, x1)
        fb = (bits >> np.uint32(9)) | _ONE_BITS
        u = lax.bitcast_convert_type(fb, jnp.float32) - jnp.float32(1.0)
        bern_ref[...] = (u < p).astype(jnp.float32)


def kernel(adjacency_probability_matrix, adjacency_temperature, nor_probability, nor_temperature):
    key = jax.random.key(42)
    k1, k2 = jax.random.split(key)
    keys = jnp.concatenate(
        [jax.random.key_data(k1), jax.random.key_data(k2)]).astype(jnp.uint32)
    temps = jnp.stack([jnp.asarray(adjacency_temperature, jnp.float32),
                       jnp.asarray(nor_temperature, jnp.float32)])

    samples, bern = pl.pallas_call(
        _sample_kernel,
        grid=(NUM_OUT // ROWS,),
        in_specs=[
            pl.BlockSpec(memory_space=pltpu.SMEM),
            pl.BlockSpec(memory_space=pltpu.SMEM),
            pl.BlockSpec((ROWS, NUM_IN), lambda i: (i, 0)),
            pl.BlockSpec((8, 128), lambda i: (0, 0)),
        ],
        out_specs=[
            pl.BlockSpec((ROWS, 2), lambda i: (i, 0)),
            pl.BlockSpec((8, 128), lambda i: (0, 0)),
        ],
        out_shape=[
            jax.ShapeDtypeStruct((NUM_OUT, 2), jnp.int32),
            jax.ShapeDtypeStruct((8, 128), jnp.float32),
        ],
    )(keys, temps, adjacency_probability_matrix, nor_probability.reshape(8, 128))

    return samples.astype(jnp.int64), bern.reshape(NUM_OUT)


# speedup vs baseline: 1.0880x; 1.0880x over previous
import jax
import jax.numpy as jnp
from jax import lax
from jax.experimental import pallas as pl
from jax.experimental.pallas import tpu as pltpu
import numpy as np

NUM_IN = 100000
NUM_OUT = 1024
ROWS = 8
SLAB = NUM_OUT * NUM_IN

_TINY = np.float32(np.finfo(np.float32).tiny)
_ONE_BITS = np.uint32(np.float32(1.0).view(np.uint32))

W = 1024
NCH = NUM_IN // W
TAIL = NUM_IN - NCH * W

_ROTS = ((13, 15, 26, 6), (17, 29, 16, 24))


def _hash_bits(ks, x1):
    ks0, ks1, ks2 = ks
    x0 = ks0
    inj0 = (ks1, ks2, ks0, ks1, ks2)
    inj1 = (ks2 + np.uint32(1), ks0 + np.uint32(2), ks1 + np.uint32(3),
            ks2 + np.uint32(4), ks0 + np.uint32(5))
    for i in range(5):
        for r in _ROTS[i % 2]:
            x0 = x0 + x1
            x1 = ((x1 << np.uint32(r)) | (x1 >> np.uint32(32 - r))) ^ x0
        x0 = x0 + inj0[i]
        x1 = x1 + inj1[i]
    return x0 ^ x1


def _bits_to_open01(bits):
    fb = (bits >> np.uint32(9)) | _ONE_BITS
    f = lax.bitcast_convert_type(fb, jnp.float32) - jnp.float32(1.0)
    return jnp.maximum(_TINY, f)


def _sample_kernel(keys_ref, temp_ref, adj_ref, nor_ref, out_ref, bern_ref):
    pid = pl.program_id(0)
    t = temp_ref[0]
    k0 = keys_ref[0]
    k1 = keys_ref[1]
    ks = (k0, k1, k0 ^ k1 ^ np.uint32(0x1BD11BDA))

    row0 = (pid * ROWS).astype(jnp.uint32)
    r_u = lax.broadcasted_iota(jnp.uint32, (ROWS, 1), 0) + row0
    base0 = r_u * np.uint32(NUM_IN) + k1

    def vals_for(logits, x1c):
        out = []
        for s in range(2):
            bits = _hash_bits(ks, x1c if s == 0 else x1c + np.uint32(SLAB))
            u = _bits_to_open01(bits)
            out.append(logits - jnp.log(-jnp.log(u)))
        return out

    NPAIR = NCH // 2

    def body(j, carry):
        x1c, (av0, aj0, av1, aj1) = carry
        ja = j * 2
        la = adj_ref[:, pl.ds(ja * W, W)] * t
        lb = adj_ref[:, pl.ds(ja * W + W, W)] * t
        va0, va1 = vals_for(la, x1c)
        vb0, vb1 = vals_for(lb, x1c + np.uint32(W))
        pb0 = vb0 > va0
        pb1 = vb1 > va1
        v0 = jnp.where(pb0, vb0, va0)
        j0 = jnp.where(pb0, ja + 1, ja)
        v1 = jnp.where(pb1, vb1, va1)
        j1 = jnp.where(pb1, ja + 1, ja)
        up0 = v0 > av0
        up1 = v1 > av1
        av0 = jnp.where(up0, v0, av0)
        aj0 = jnp.where(up0, j0, aj0)
        av1 = jnp.where(up1, v1, av1)
        aj1 = jnp.where(up1, j1, aj1)
        return x1c + np.uint32(2 * W), (av0, aj0, av1, aj1)

    c_w = lax.broadcasted_iota(jnp.uint32, (ROWS, W), 1)
    neg_inf = jnp.full((ROWS, W), -jnp.inf, jnp.float32)
    zeros_i = jnp.zeros((ROWS, W), jnp.int32)
    _, (av0, aj0, av1, aj1) = lax.fori_loop(
        0, NPAIR, body,
        (base0 + c_w, (neg_inf, zeros_i, neg_inf, zeros_i)))

    X0 = NPAIR * 2 * W
    x_logits = adj_ref[:, pl.ds(X0, W)] * t
    xv0, xv1 = vals_for(x_logits, base0 + c_w + np.uint32(X0))

    c_t = lax.broadcasted_iota(jnp.uint32, (ROWS, TAIL), 1)
    t_logits = adj_ref[:, pl.ds(NCH * W, TAIL)] * t
    tv0, tv1 = vals_for(t_logits, base0 + c_t + np.uint32(NCH * W))

    lane_w = lax.broadcasted_iota(jnp.int32, (ROWS, W), 1)
    lane_t = lax.broadcasted_iota(jnp.int32, (ROWS, TAIL), 1)
    sentinel = jnp.int32(np.iinfo(np.int32).max)

    cols = []
    for av, aj, xv, tv in ((av0, aj0, xv0, tv0), (av1, aj1, xv1, tv1)):
        m_main = jnp.max(av, axis=1, keepdims=True)
        c_main = aj * W + lane_w
        i_main = jnp.min(jnp.where(av == m_main, c_main, sentinel), axis=1)
        m_x = jnp.max(xv, axis=1, keepdims=True)
        c_x = lane_w + jnp.int32(X0)
        i_x = jnp.min(jnp.where(xv == m_x, c_x, sentinel), axis=1)
        m_tail = jnp.max(tv, axis=1, keepdims=True)
        c_tail = lane_t + jnp.int32(NCH * W)
        i_tail = jnp.min(jnp.where(tv == m_tail, c_tail, sentinel), axis=1)
        xw = m_tail[:, 0] > m_x[:, 0]
        m_xt = jnp.where(xw, m_tail[:, 0], m_x[:, 0])
        i_xt = jnp.where(xw, i_tail, i_x)
        tw = m_xt > m_main[:, 0]
        cols.append(jnp.where(tw, i_xt, i_main)[:, None])
    out_ref[...] = jnp.concatenate(cols, axis=1)

    @pl.when(pid == 0)
    def _():
        tb = temp_ref[1]
        kb0 = keys_ref[2]
        kb1 = keys_ref[3]
        kb = (kb0, kb1, kb0 ^ kb1 ^ np.uint32(0x1BD11BDA))
        p = jax.nn.sigmoid(nor_ref[...] * tb)
        a = lax.broadcasted_iota(jnp.uint32, (8, 128), 0)
        b = lax.broadcasted_iota(jnp.uint32, (8, 128), 1)
        x1 = a * np.uint32(128) + b + kb1
        bits = _hash_bits(kb, x1)
        fb = (bits >> np.uint32(9)) | _ONE_BITS
        u = lax.bitcast_convert_type(fb, jnp.float32) - jnp.float32(1.0)
        bern_ref[...] = (u < p).astype(jnp.float32)


def kernel(adjacency_probability_matrix, adjacency_temperature, nor_probability, nor_temperature):
    key = jax.random.key(42)
    k1, k2 = jax.random.split(key)
    keys = jnp.concatenate(
        [jax.random.key_data(k1), jax.random.key_data(k2)]).astype(jnp.uint32)
    temps = jnp.stack([jnp.asarray(adjacency_temperature, jnp.float32),
                       jnp.asarray(nor_temperature, jnp.float32)])

    samples, bern = pl.pallas_call(
        _sample_kernel,
        grid=(NUM_OUT // ROWS,),
        in_specs=[
            pl.BlockSpec(memory_space=pltpu.SMEM),
            pl.BlockSpec(memory_space=pltpu.SMEM),
            pl.BlockSpec((ROWS, NUM_IN), lambda i: (i, 0)),
            pl.BlockSpec((8, 128), lambda i: (0, 0)),
        ],
        out_specs=[
            pl.BlockSpec((ROWS, 2), lambda i: (i, 0)),
            pl.BlockSpec((8, 128), lambda i: (0, 0)),
        ],
        out_shape=[
            jax.ShapeDtypeStruct((NUM_OUT, 2), jnp.int32),
            jax.ShapeDtypeStruct((8, 128), jnp.float32),
        ],
    )(keys, temps, adjacency_probability_matrix, nor_probability.reshape(8, 128))

    return samples.astype(jnp.int64), bern.reshape(NUM_OUT)


# speedup vs baseline: 1.0926x; 1.0043x over previous
import jax
import jax.numpy as jnp
from jax import lax
from jax.experimental import pallas as pl
from jax.experimental.pallas import tpu as pltpu
import numpy as np

NUM_IN = 100000
NUM_OUT = 1024
ROWS = 8
SLAB = NUM_OUT * NUM_IN

_TINY = np.float32(np.finfo(np.float32).tiny)
_ONE_BITS = np.uint32(np.float32(1.0).view(np.uint32))

W = 1024
NCH = NUM_IN // W
TAIL = NUM_IN - NCH * W

_ROTS = ((13, 15, 26, 6), (17, 29, 16, 24))


def _hash_bits(ks, x1):
    ks0, ks1, ks2 = ks
    x0 = ks0
    inj0 = (ks1, ks2, ks0, ks1, ks2)
    inj1 = (ks2 + np.uint32(1), ks0 + np.uint32(2), ks1 + np.uint32(3),
            ks2 + np.uint32(4), ks0 + np.uint32(5))
    for i in range(5):
        for r in _ROTS[i % 2]:
            x0 = x0 + x1
            x1 = ((x1 << np.uint32(r)) | (x1 >> np.uint32(32 - r))) ^ x0
        x0 = x0 + inj0[i]
        x1 = x1 + inj1[i]
    return x0 ^ x1


def _bits_to_open01(bits):
    fb = (bits >> np.uint32(9)) | _ONE_BITS
    f = lax.bitcast_convert_type(fb, jnp.float32) - jnp.float32(1.0)
    return jnp.maximum(_TINY, f)


def _sample_kernel(keys_ref, temp_ref, adj_ref, nor_ref, out_ref, bern_ref):
    pid = pl.program_id(0)
    t = temp_ref[0]
    k0 = keys_ref[0]
    k1 = keys_ref[1]
    ks = (k0, k1, k0 ^ k1 ^ np.uint32(0x1BD11BDA))

    row0 = (pid * ROWS).astype(jnp.uint32)
    r_u = lax.broadcasted_iota(jnp.uint32, (ROWS, 1), 0) + row0
    base0 = r_u * np.uint32(NUM_IN) + k1

    def vals_for(logits, x1c):
        out = []
        for s in range(2):
            bits = _hash_bits(ks, x1c if s == 0 else x1c + np.uint32(SLAB))
            u = _bits_to_open01(bits)
            out.append(logits - jnp.log(-jnp.log(u)))
        return out

    GRP = 4
    NGRP = NCH // GRP

    def combine(pa, pb):
        va, ja_, sa0 = pa
        vb, jb_, sb0 = pb
        up = vb > va
        return jnp.where(up, vb, va), jnp.where(up, jb_, ja_), None

    def body(j, carry):
        x1c, (av0, aj0, av1, aj1) = carry
        ja = j * GRP
        pairs0 = []
        pairs1 = []
        for uu in range(GRP):
            lg = adj_ref[:, pl.ds((ja + uu) * W, W)] * t
            w0, w1 = vals_for(lg, x1c + np.uint32(uu * W))
            pairs0.append((w0, ja + uu, None))
            pairs1.append((w1, ja + uu, None))
        while len(pairs0) > 1:
            pairs0 = [combine(pairs0[i], pairs0[i + 1])
                      for i in range(0, len(pairs0), 2)]
            pairs1 = [combine(pairs1[i], pairs1[i + 1])
                      for i in range(0, len(pairs1), 2)]
        v0, j0, _ = pairs0[0]
        v1, j1, _ = pairs1[0]
        up0 = v0 > av0
        up1 = v1 > av1
        av0 = jnp.where(up0, v0, av0)
        aj0 = jnp.where(up0, j0, aj0)
        av1 = jnp.where(up1, v1, av1)
        aj1 = jnp.where(up1, j1, aj1)
        return x1c + np.uint32(GRP * W), (av0, aj0, av1, aj1)

    c_w = lax.broadcasted_iota(jnp.uint32, (ROWS, W), 1)
    neg_inf = jnp.full((ROWS, W), -jnp.inf, jnp.float32)
    zeros_i = jnp.zeros((ROWS, W), jnp.int32)
    _, (av0, aj0, av1, aj1) = lax.fori_loop(
        0, NGRP, body,
        (base0 + c_w, (neg_inf, zeros_i, neg_inf, zeros_i)))

    X0 = NGRP * GRP * W
    x_logits = adj_ref[:, pl.ds(X0, W)] * t
    xv0, xv1 = vals_for(x_logits, base0 + c_w + np.uint32(X0))

    c_t = lax.broadcasted_iota(jnp.uint32, (ROWS, TAIL), 1)
    t_logits = adj_ref[:, pl.ds(NCH * W, TAIL)] * t
    tv0, tv1 = vals_for(t_logits, base0 + c_t + np.uint32(NCH * W))

    lane_w = lax.broadcasted_iota(jnp.int32, (ROWS, W), 1)
    lane_t = lax.broadcasted_iota(jnp.int32, (ROWS, TAIL), 1)
    sentinel = jnp.int32(np.iinfo(np.int32).max)

    cols = []
    for av, aj, xv, tv in ((av0, aj0, xv0, tv0), (av1, aj1, xv1, tv1)):
        m_main = jnp.max(av, axis=1, keepdims=True)
        c_main = aj * W + lane_w
        i_main = jnp.min(jnp.where(av == m_main, c_main, sentinel), axis=1)
        m_x = jnp.max(xv, axis=1, keepdims=True)
        c_x = lane_w + jnp.int32(X0)
        i_x = jnp.min(jnp.where(xv == m_x, c_x, sentinel), axis=1)
        m_tail = jnp.max(tv, axis=1, keepdims=True)
        c_tail = lane_t + jnp.int32(NCH * W)
        i_tail = jnp.min(jnp.where(tv == m_tail, c_tail, sentinel), axis=1)
        xw = m_tail[:, 0] > m_x[:, 0]
        m_xt = jnp.where(xw, m_tail[:, 0], m_x[:, 0])
        i_xt = jnp.where(xw, i_tail, i_x)
        tw = m_xt > m_main[:, 0]
        cols.append(jnp.where(tw, i_xt, i_main)[:, None])
    out_ref[...] = jnp.concatenate(cols, axis=1)

    @pl.when(pid == 0)
    def _():
        tb = temp_ref[1]
        kb0 = keys_ref[2]
        kb1 = keys_ref[3]
        kb = (kb0, kb1, kb0 ^ kb1 ^ np.uint32(0x1BD11BDA))
        p = jax.nn.sigmoid(nor_ref[...] * tb)
        a = lax.broadcasted_iota(jnp.uint32, (8, 128), 0)
        b = lax.broadcasted_iota(jnp.uint32, (8, 128), 1)
        x1 = a * np.uint32(128) + b + kb1
        bits = _hash_bits(kb, x1)
        fb = (bits >> np.uint32(9)) | _ONE_BITS
        u = lax.bitcast_convert_type(fb, jnp.float32) - jnp.float32(1.0)
        bern_ref[...] = (u < p).astype(jnp.float32)


def kernel(adjacency_probability_matrix, adjacency_temperature, nor_probability, nor_temperature):
    key = jax.random.key(42)
    k1, k2 = jax.random.split(key)
    keys = jnp.concatenate(
        [jax.random.key_data(k1), jax.random.key_data(k2)]).astype(jnp.uint32)
    temps = jnp.stack([jnp.asarray(adjacency_temperature, jnp.float32),
                       jnp.asarray(nor_temperature, jnp.float32)])

    samples, bern = pl.pallas_call(
        _sample_kernel,
        grid=(NUM_OUT // ROWS,),
        in_specs=[
            pl.BlockSpec(memory_space=pltpu.SMEM),
            pl.BlockSpec(memory_space=pltpu.SMEM),
            pl.BlockSpec((ROWS, NUM_IN), lambda i: (i, 0)),
            pl.BlockSpec((8, 128), lambda i: (0, 0)),
        ],
        out_specs=[
            pl.BlockSpec((ROWS, 2), lambda i: (i, 0)),
            pl.BlockSpec((8, 128), lambda i: (0, 0)),
        ],
        out_shape=[
            jax.ShapeDtypeStruct((NUM_OUT, 2), jnp.int32),
            jax.ShapeDtypeStruct((8, 128), jnp.float32),
        ],
    )(keys, temps, adjacency_probability_matrix, nor_probability.reshape(8, 128))

    return samples.astype(jnp.int64), bern.reshape(NUM_OUT)


# speedup vs baseline: 1.0972x; 1.0042x over previous
import jax
import jax.numpy as jnp
from jax import lax
from jax.experimental import pallas as pl
from jax.experimental.pallas import tpu as pltpu
import numpy as np

NUM_IN = 100000
NUM_OUT = 1024
ROWS = 8
SLAB = NUM_OUT * NUM_IN

_TINY = np.float32(np.finfo(np.float32).tiny)
_ONE_BITS = np.uint32(np.float32(1.0).view(np.uint32))

W = 1024
NCH = NUM_IN // W
TAIL = NUM_IN - NCH * W

_ROTS = ((13, 15, 26, 6), (17, 29, 16, 24))


def _hash_bits(ks, x1):
    ks0, ks1, ks2 = ks
    x0 = ks0
    inj0 = (ks1, ks2, ks0, ks1, ks2)
    inj1 = (ks2 + np.uint32(1), ks0 + np.uint32(2), ks1 + np.uint32(3),
            ks2 + np.uint32(4), ks0 + np.uint32(5))
    for i in range(5):
        for r in _ROTS[i % 2]:
            x0 = x0 + x1
            x1 = ((x1 << np.uint32(r)) | (x1 >> np.uint32(32 - r))) ^ x0
        x0 = x0 + inj0[i]
        x1 = x1 + inj1[i]
    return x0 ^ x1


def _bits_to_open01(bits):
    fb = (bits >> np.uint32(9)) | _ONE_BITS
    f = lax.bitcast_convert_type(fb, jnp.float32) - jnp.float32(1.0)
    return jnp.maximum(_TINY, f)


def _sample_kernel(keys_ref, temp_ref, adj_ref, nor_ref, out_ref, bern_ref):
    pid = pl.program_id(0)
    t = temp_ref[0]
    k0 = keys_ref[0]
    k1 = keys_ref[1]
    ks = (k0, k1, k0 ^ k1 ^ np.uint32(0x1BD11BDA))

    row0 = (pid * ROWS).astype(jnp.uint32)
    r_u = lax.broadcasted_iota(jnp.uint32, (ROWS, 1), 0) + row0
    base0 = r_u * np.uint32(NUM_IN) + k1

    def vals_for(logits, x1c):
        out = []
        for s in range(2):
            bits = _hash_bits(ks, x1c if s == 0 else x1c + np.uint32(SLAB))
            u = _bits_to_open01(bits)
            out.append(logits - jnp.log(-jnp.log(u)))
        return out

    GRP = 8
    NGRP = NCH // GRP

    def combine(pa, pb):
        va, ja_, sa0 = pa
        vb, jb_, sb0 = pb
        up = vb > va
        return jnp.where(up, vb, va), jnp.where(up, jb_, ja_), None

    def body(j, carry):
        x1c, (av0, aj0, av1, aj1) = carry
        ja = j * GRP
        pairs0 = []
        pairs1 = []
        for uu in range(GRP):
            lg = adj_ref[:, pl.ds((ja + uu) * W, W)] * t
            w0, w1 = vals_for(lg, x1c + np.uint32(uu * W))
            pairs0.append((w0, ja + uu, None))
            pairs1.append((w1, ja + uu, None))
        while len(pairs0) > 1:
            pairs0 = [combine(pairs0[i], pairs0[i + 1])
                      for i in range(0, len(pairs0), 2)]
            pairs1 = [combine(pairs1[i], pairs1[i + 1])
                      for i in range(0, len(pairs1), 2)]
        v0, j0, _ = pairs0[0]
        v1, j1, _ = pairs1[0]
        up0 = v0 > av0
        up1 = v1 > av1
        av0 = jnp.where(up0, v0, av0)
        aj0 = jnp.where(up0, j0, aj0)
        av1 = jnp.where(up1, v1, av1)
        aj1 = jnp.where(up1, j1, aj1)
        return x1c + np.uint32(GRP * W), (av0, aj0, av1, aj1)

    c_w = lax.broadcasted_iota(jnp.uint32, (ROWS, W), 1)
    neg_inf = jnp.full((ROWS, W), -jnp.inf, jnp.float32)
    zeros_i = jnp.zeros((ROWS, W), jnp.int32)
    _, (av0, aj0, av1, aj1) = lax.fori_loop(
        0, NGRP, body,
        (base0 + c_w, (neg_inf, zeros_i, neg_inf, zeros_i)))

    X0 = NGRP * GRP * W
    x_logits = adj_ref[:, pl.ds(X0, W)] * t
    xv0, xv1 = vals_for(x_logits, base0 + c_w + np.uint32(X0))

    c_t = lax.broadcasted_iota(jnp.uint32, (ROWS, TAIL), 1)
    t_logits = adj_ref[:, pl.ds(NCH * W, TAIL)] * t
    tv0, tv1 = vals_for(t_logits, base0 + c_t + np.uint32(NCH * W))

    lane_w = lax.broadcasted_iota(jnp.int32, (ROWS, W), 1)
    lane_t = lax.broadcasted_iota(jnp.int32, (ROWS, TAIL), 1)
    sentinel = jnp.int32(np.iinfo(np.int32).max)

    cols = []
    for av, aj, xv, tv in ((av0, aj0, xv0, tv0), (av1, aj1, xv1, tv1)):
        m_main = jnp.max(av, axis=1, keepdims=True)
        c_main = aj * W + lane_w
        i_main = jnp.min(jnp.where(av == m_main, c_main, sentinel), axis=1)
        m_x = jnp.max(xv, axis=1, keepdims=True)
        c_x = lane_w + jnp.int32(X0)
        i_x = jnp.min(jnp.where(xv == m_x, c_x, sentinel), axis=1)
        m_tail = jnp.max(tv, axis=1, keepdims=True)
        c_tail = lane_t + jnp.int32(NCH * W)
        i_tail = jnp.min(jnp.where(tv == m_tail, c_tail, sentinel), axis=1)
        xw = m_tail[:, 0] > m_x[:, 0]
        m_xt = jnp.where(xw, m_tail[:, 0], m_x[:, 0])
        i_xt = jnp.where(xw, i_tail, i_x)
        tw = m_xt > m_main[:, 0]
        cols.append(jnp.where(tw, i_xt, i_main)[:, None])
    out_ref[...] = jnp.concatenate(cols, axis=1)

    @pl.when(pid == 0)
    def _():
        tb = temp_ref[1]
        kb0 = keys_ref[2]
        kb1 = keys_ref[3]
        kb = (kb0, kb1, kb0 ^ kb1 ^ np.uint32(0x1BD11BDA))
        p = jax.nn.sigmoid(nor_ref[...] * tb)
        a = lax.broadcasted_iota(jnp.uint32, (8, 128), 0)
        b = lax.broadcasted_iota(jnp.uint32, (8, 128), 1)
        x1 = a * np.uint32(128) + b + kb1
        bits = _hash_bits(kb, x1)
        fb = (bits >> np.uint32(9)) | _ONE_BITS
        u = lax.bitcast_convert_type(fb, jnp.float32) - jnp.float32(1.0)
        bern_ref[...] = (u < p).astype(jnp.float32)


def kernel(adjacency_probability_matrix, adjacency_temperature, nor_probability, nor_temperature):
    key = jax.random.key(42)
    k1, k2 = jax.random.split(key)
    keys = jnp.concatenate(
        [jax.random.key_data(k1), jax.random.key_data(k2)]).astype(jnp.uint32)
    temps = jnp.stack([jnp.asarray(adjacency_temperature, jnp.float32),
                       jnp.asarray(nor_temperature, jnp.float32)])

    samples, bern = pl.pallas_call(
        _sample_kernel,
        grid=(NUM_OUT // ROWS,),
        in_specs=[
            pl.BlockSpec(memory_space=pltpu.SMEM),
            pl.BlockSpec(memory_space=pltpu.SMEM),
            pl.BlockSpec((ROWS, NUM_IN), lambda i: (i, 0)),
            pl.BlockSpec((8, 128), lambda i: (0, 0)),
        ],
        out_specs=[
            pl.BlockSpec((ROWS, 2), lambda i: (i, 0)),
            pl.BlockSpec((8, 128), lambda i: (0, 0)),
        ],
        out_shape=[
            jax.ShapeDtypeStruct((NUM_OUT, 2), jnp.int32),
            jax.ShapeDtypeStruct((8, 128), jnp.float32),
        ],
    )(keys, temps, adjacency_probability_matrix, nor_probability.reshape(8, 128))

    return samples.astype(jnp.int64), bern.reshape(NUM_OUT)
